# baseline (speedup 1.0000x reference)
import functools

import jax
import jax.numpy as jnp
from jax.experimental import pallas as pl
from jax.experimental.pallas import tpu as pltpu

_ROWS = 256
_TCHUNK = 16


def _ceil_to(n, m):
    return ((n + m - 1) // m) * m


def _lstm_tile(places_ref, x_ref, wih_ref, whh_ref, b_ref, out_ref,
               gin_ref, h_ref, c_ref, o_ref, *, hidden, n_chunks):
    Bt, C, D = x_ref.shape
    H = hidden
    k = pl.program_id(1)

    whh = whh_ref[...]
    bias = b_ref[...]
    places = places_ref[...]

    @pl.when(k == 0)
    def _init():
        h_ref[...] = jnp.zeros_like(h_ref)
        c_ref[...] = jnp.zeros_like(c_ref)
        o_ref[...] = jnp.zeros_like(o_ref)

    xt = jnp.swapaxes(x_ref[...], 0, 1).reshape(C * Bt, D)
    gin_ref[...] = (
        jnp.dot(xt.astype(jnp.bfloat16), wih_ref[...],
                preferred_element_type=jnp.float32) + bias)

    def step(t, row, st):
        h, c, out = st
        gates = (jnp.dot(h, whh, preferred_element_type=jnp.float32)
                 + gin_ref[pl.ds(row, Bt), :])
        act = jnp.tanh(gates[:, 0:3 * H]) * 0.5 + 0.5
        i_g = act[:, 0 * H:1 * H]
        f_g = act[:, 1 * H:2 * H]
        o_g = act[:, 2 * H:3 * H]
        g_g = jnp.tanh(gates[:, 3 * H:4 * H])
        c = f_g * c + i_g * g_g
        h = o_g * jnp.tanh(c)
        out = jnp.where(places == t, h, out)
        return h, c, out

    st = (h_ref[...], c_ref[...], o_ref[...])
    for j in range(C):
        st = step(k * C + j, pl.multiple_of(j * Bt, Bt), st)

    h_ref[...], c_ref[...], o_ref[...] = st
    out_ref[...] = st[2]


@jax.jit
def kernel(x, real_positions, wih_packed, whh_packed, bias_packed):
    B, S, D = x.shape
    H, Gp = whh_packed.shape
    Bt = _ROWS
    Bp = _ceil_to(B, Bt)
    C = _TCHUNK if S % _TCHUNK == 0 else S
    n_chunks = S // C

    x_p = x
    if Bp != B:
        x_p = jnp.pad(x, ((0, Bp - B), (0, 0), (0, 0)))

    sv = jnp.concatenate(
        [jnp.full((3 * H,), 0.5, jnp.float32),
         jnp.ones((Gp - 3 * H,), jnp.float32)])[None, :]
    whh_s = whh_packed * sv
    wih_s = wih_packed * sv.astype(wih_packed.dtype)
    bias_s = bias_packed * sv

    lengths = jnp.sum(real_positions.astype(jnp.float32), axis=-1)
    places = lengths.astype(jnp.int32) - 1
    places = jnp.where(places < 0, places + S, places)[:, None]
    if Bp != B:
        places = jnp.pad(places, ((0, Bp - B), (0, 0)))

    out = pl.pallas_call(
        functools.partial(_lstm_tile, hidden=H, n_chunks=n_chunks),
        out_shape=jax.ShapeDtypeStruct((Bp, H), jnp.float32),
        grid_spec=pltpu.PrefetchScalarGridSpec(
            num_scalar_prefetch=0,
            grid=(Bp // Bt, n_chunks),
            in_specs=[
                pl.BlockSpec((Bt, 1), lambda g, k: (g, 0)),
                pl.BlockSpec((Bt, C, D), lambda g, k: (g, k, 0)),
                pl.BlockSpec((D, Gp), lambda g, k: (0, 0)),
                pl.BlockSpec((H, Gp), lambda g, k: (0, 0)),
                pl.BlockSpec((1, Gp), lambda g, k: (0, 0)),
            ],
            out_specs=pl.BlockSpec((Bt, H), lambda g, k: (g, 0)),
            scratch_shapes=[
                pltpu.VMEM((C * Bt, Gp), jnp.float32),
                pltpu.VMEM((Bt, H), jnp.float32),
                pltpu.VMEM((Bt, H), jnp.float32),
                pltpu.VMEM((Bt, H), jnp.float32),
            ],
        ),
        compiler_params=pltpu.CompilerParams(
            dimension_semantics=("parallel", "arbitrary")),
    )(places, x_p, wih_s, whh_s, bias_s)

    return out[:B]


# speedup vs baseline: 1.0702x; 1.0702x over previous
import functools

import jax
import jax.numpy as jnp
from jax.experimental import pallas as pl
from jax.experimental.pallas import tpu as pltpu

_ROWS = 256
_TCHUNK = 16


def _ceil_to(n, m):
    return ((n + m - 1) // m) * m


def _lstm_tile(places_ref, x_ref, wih_ref, whh_ref, b_ref, out_ref,
               gin_ref, h_ref, c_ref, o_ref, *, hidden, n_chunks):
    Bt, C, D = x_ref.shape
    H = hidden
    k = pl.program_id(1)

    whh = whh_ref[...]
    bias = b_ref[...]
    places = places_ref[...]

    @pl.when(k == 0)
    def _init():
        h_ref[...] = jnp.zeros_like(h_ref)
        c_ref[...] = jnp.zeros_like(c_ref)
        o_ref[...] = jnp.zeros_like(o_ref)

    xt = jnp.swapaxes(x_ref[...], 0, 1).reshape(C * Bt, D)
    gin_ref[...] = (
        jnp.dot(xt.astype(jnp.bfloat16), wih_ref[...],
                preferred_element_type=jnp.float32) + bias)

    def step(t, row, st):
        h, c, out = st
        gates = (jnp.dot(h, whh, preferred_element_type=jnp.float32)
                 + gin_ref[pl.ds(row, Bt), :])
        act = jax.nn.sigmoid(gates[:, 0:3 * H])
        i_g = act[:, 0 * H:1 * H]
        f_g = act[:, 1 * H:2 * H]
        o_g = act[:, 2 * H:3 * H]
        g_g = jnp.tanh(gates[:, 3 * H:4 * H])
        c = f_g * c + i_g * g_g
        h = o_g * jnp.tanh(c)
        out = jnp.where(places == t, h, out)
        return h, c, out

    st = (h_ref[...], c_ref[...], o_ref[...])
    for j in range(C):
        st = step(k * C + j, pl.multiple_of(j * Bt, Bt), st)

    h_ref[...], c_ref[...], o_ref[...] = st
    out_ref[...] = st[2]


@jax.jit
def kernel(x, real_positions, wih_packed, whh_packed, bias_packed):
    B, S, D = x.shape
    H, Gp = whh_packed.shape
    Bt = _ROWS
    Bp = _ceil_to(B, Bt)
    C = _TCHUNK if S % _TCHUNK == 0 else S
    n_chunks = S // C

    x_p = x
    if Bp != B:
        x_p = jnp.pad(x, ((0, Bp - B), (0, 0), (0, 0)))

    lengths = jnp.sum(real_positions.astype(jnp.float32), axis=-1)
    places = lengths.astype(jnp.int32) - 1
    places = jnp.where(places < 0, places + S, places)[:, None]
    if Bp != B:
        places = jnp.pad(places, ((0, Bp - B), (0, 0)))

    out = pl.pallas_call(
        functools.partial(_lstm_tile, hidden=H, n_chunks=n_chunks),
        out_shape=jax.ShapeDtypeStruct((Bp, H), jnp.float32),
        grid_spec=pltpu.PrefetchScalarGridSpec(
            num_scalar_prefetch=0,
            grid=(Bp // Bt, n_chunks),
            in_specs=[
                pl.BlockSpec((Bt, 1), lambda g, k: (g, 0)),
                pl.BlockSpec((Bt, C, D), lambda g, k: (g, k, 0)),
                pl.BlockSpec((D, Gp), lambda g, k: (0, 0)),
                pl.BlockSpec((H, Gp), lambda g, k: (0, 0)),
                pl.BlockSpec((1, Gp), lambda g, k: (0, 0)),
            ],
            out_specs=pl.BlockSpec((Bt, H), lambda g, k: (g, 0)),
            scratch_shapes=[
                pltpu.VMEM((C * Bt, Gp), jnp.float32),
                pltpu.VMEM((Bt, H), jnp.float32),
                pltpu.VMEM((Bt, H), jnp.float32),
                pltpu.VMEM((Bt, H), jnp.float32),
            ],
        ),
        compiler_params=pltpu.CompilerParams(
            dimension_semantics=("parallel", "arbitrary")),
    )(places, x_p, wih_packed, whh_packed, bias_packed)

    return out[:B]
